# baseline (speedup 1.0000x reference)
import functools

import jax
import jax.numpy as jnp
from jax import lax
from jax.experimental import pallas as pl
from jax.experimental.pallas import tpu as pltpu
from jax.experimental.pallas import tpu_sc as plsc

N = 100000
NPAD = 100096
E = 1600000
HID = 32
HALF = 16
NUM_CLASSES = 3

SUB = 128
RPB = 2
ROWS = 12800
PAD_E = ROWS * SUB
TILE_ROWS = ROWS // 16
BLOCKS = TILE_ROWS // RPB
NPT = NPAD // 16
ZCH = 128

NB = NPT
GRID = NPAD // NB


def _make_edge_body():
    def body(za_h, zb_h, src_h, dst_h, ew_h, ua_h, ub_h,
             sb0, sb1, sb2, sb3, sb4, db0, db1, db2, db3, db4,
             eb0, eb1, eb2, eb3, eb4,
             r0, r1, r2, r3, r4, acc,
             g0, g1, g2, g3, g4, s0, s1, s2, s3, s4):
        c = lax.axis_index("c")
        s = lax.axis_index("s")
        B = ((sb0, db0, eb0, r0, g0, s0), (sb1, db1, eb1, r1, g1, s1),
             (sb2, db2, eb2, r2, g2, s2), (sb3, db3, eb3, r3, g3, s3),
             (sb4, db4, eb4, r4, g4, s4))

        def zrow(i, _):
            r0[i, :] = jnp.zeros((HALF,), jnp.float32)
            return 0
        lax.fori_loop(0, RPB * SUB, zrow, 0)
        nz = NPT // ZCH
        for k in range(nz):
            pltpu.sync_copy(r0.at[pl.ds(0, ZCH)], acc.at[pl.ds(s * NPT + k * ZCH, ZCH)])
        rem = NPT - nz * ZCH
        if rem:
            pltpu.sync_copy(r0.at[pl.ds(0, rem)], acc.at[pl.ds(s * NPT + nz * ZCH, rem)])
        plsc.subcore_barrier()

        def stage(bi, blk):
            sb, db, eb = B[bi][0], B[bi][1], B[bi][2]
            rowbase = s * TILE_ROWS + blk * RPB
            pltpu.sync_copy(src_h.at[pl.ds(rowbase, RPB)], sb)
            pltpu.sync_copy(dst_h.at[pl.ds(rowbase, RPB)], db)
            pltpu.sync_copy(ew_h.at[pl.ds(rowbase, RPB)], eb)

        def fire_g(bi):
            sb, rows, sem = B[bi][0], B[bi][3], B[bi][4]

            @pl.when(c == 0)
            def _():
                for j in range(RPB):
                    pltpu.async_copy(za_h.at[sb.at[j]], rows.at[pl.ds(j * SUB, SUB)], sem)

            @pl.when(c == 1)
            def _():
                for j in range(RPB):
                    pltpu.async_copy(zb_h.at[sb.at[j]], rows.at[pl.ds(j * SUB, SUB)], sem)

        def drain(sem, rows):
            pltpu.make_async_copy(za_h.at[pl.ds(0, RPB * SUB)], rows, sem).wait()

        def scale(bi):
            eb, rows = B[bi][2], B[bi][3]
            for j in range(RPB):
                def s16(t, _):
                    w16 = eb[j, pl.ds(t * 16, 16)]
                    base = j * SUB + t * 16
                    for l in range(16):
                        rows[base + l, :] = rows[base + l, :] * w16[l]
                    return 0
                lax.fori_loop(0, SUB // 16, s16, 0)

        def fire_s(bi):
            db, rows, sem = B[bi][1], B[bi][3], B[bi][5]
            for j in range(RPB):
                pltpu.async_copy(rows.at[pl.ds(j * SUB, SUB)], acc.at[db.at[j]], sem,
                                 add=True)

        for b0 in range(3):
            stage(b0, b0)
            fire_g(b0)

        def quint(i, _):
            for u in range(5):
                blk = i * 5 + u
                bl = (u + 3) % 5

                @pl.when(blk + 3 < BLOCKS)
                def _():
                    @pl.when(blk >= 2)
                    def _():
                        drain(B[bl][5], B[bl][3])
                    stage(bl, blk + 3)
                    fire_g(bl)

                drain(B[u][4], B[u][3])
                scale(u)
                fire_s(u)
            return 0

        lax.fori_loop(0, BLOCKS // 5, quint, 0)

        for b0 in range(5):
            drain(B[b0][5], B[b0][3])
        plsc.subcore_barrier()

        @pl.when(c == 0)
        def _():
            pltpu.sync_copy(acc.at[pl.ds(s * NPT, NPT)], ua_h.at[pl.ds(s * NPT, NPT)])

        @pl.when(c == 1)
        def _():
            pltpu.sync_copy(acc.at[pl.ds(s * NPT, NPT)], ub_h.at[pl.ds(s * NPT, NPT)])

    return body


def _edge_call(za, zb, src2, dst2, ew2):
    f = pl.kernel(
        _make_edge_body(),
        mesh=plsc.VectorSubcoreMesh(core_axis_name="c", subcore_axis_name="s"),
        compiler_params=pltpu.CompilerParams(use_tc_tiling_on_sc=False),
        out_type=(jax.ShapeDtypeStruct((NPAD, HALF), jnp.float32),
                  jax.ShapeDtypeStruct((NPAD, HALF), jnp.float32)),
        scratch_types=(
            [pltpu.VMEM((RPB, SUB), jnp.int32)] * 10
            + [pltpu.VMEM((RPB, SUB), jnp.float32)] * 5
            + [pltpu.VMEM((RPB * SUB, HALF), jnp.float32)] * 5
            + [pltpu.VMEM_SHARED((NPAD, HALF), jnp.float32)]
            + [pltpu.SemaphoreType.DMA] * 10
        ),
    )
    return f(za, zb, src2, dst2, ew2)


@jax.jit
def _edge_pass(za, zb, src2, dst2, ew2):
    return _edge_call(za, zb, src2, dst2, ew2)


def _dinv_of(udeg_blk):
    return lax.rsqrt(udeg_blk[:, 0:1] + 1.0)


def _layer0_body(x_ref, w_ref, udeg_ref, za_ref, zb_ref):
    dv = _dinv_of(udeg_ref[...])
    xw = jnp.dot(x_ref[...], w_ref[...], preferred_element_type=jnp.float32)
    z = xw * dv
    za_ref[...] = z[:, :HALF]
    zb_ref[...] = z[:, HALF:]


def _mid_body(ua_ref, ub_ref, za_ref, zb_ref, udeg_ref, bprev_ref, w_ref,
              za_o, zb_o):
    dv = _dinv_of(udeg_ref[...])
    u_plus_z = jnp.concatenate(
        [ua_ref[...] + za_ref[...], ub_ref[...] + zb_ref[...]], axis=1)
    x = jax.nn.relu(dv * u_plus_z + bprev_ref[...])
    z = dv * jnp.dot(x, w_ref[...], preferred_element_type=jnp.float32)
    za_o[...] = z[:, :HALF]
    zb_o[...] = z[:, HALF:]


def _fin_body(ua_ref, ub_ref, za_ref, zb_ref, udeg_ref, bprev_ref, sum_ref):
    dv = _dinv_of(udeg_ref[...])
    u_plus_z = jnp.concatenate(
        [ua_ref[...] + za_ref[...], ub_ref[...] + zb_ref[...]], axis=1)
    x = jax.nn.relu(dv * u_plus_z + bprev_ref[...])
    row = (lax.broadcasted_iota(jnp.int32, (NB, 1), 0)
           + pl.program_id(0) * NB)
    x = jnp.where(row < N, x, 0.0)

    @pl.when(pl.program_id(0) == 0)
    def _():
        sum_ref[...] = jnp.zeros_like(sum_ref)

    sum_ref[...] += jnp.sum(x, axis=0, keepdims=True)


def _node_spec():
    return pl.BlockSpec((NB, HALF), lambda i: (i, 0))


def _full_spec(shape):
    return pl.BlockSpec(shape, lambda i: tuple(0 for _ in shape))


def _tc_layer0(x0, W0, udeg):
    return pl.pallas_call(
        _layer0_body,
        grid=(GRID,),
        in_specs=[pl.BlockSpec((NB, 4), lambda i: (i, 0)),
                  _full_spec((4, HID)),
                  _node_spec()],
        out_specs=(_node_spec(), _node_spec()),
        out_shape=(jax.ShapeDtypeStruct((NPAD, HALF), jnp.float32),
                   jax.ShapeDtypeStruct((NPAD, HALF), jnp.float32)),
    )(x0, W0, udeg)


def _tc_mid(ua, ub, za, zb, udeg, b_prev, W):
    return pl.pallas_call(
        _mid_body,
        grid=(GRID,),
        in_specs=[_node_spec(), _node_spec(), _node_spec(), _node_spec(),
                  _node_spec(),
                  _full_spec((1, HID)),
                  _full_spec((HID, HID))],
        out_specs=(_node_spec(), _node_spec()),
        out_shape=(jax.ShapeDtypeStruct((NPAD, HALF), jnp.float32),
                   jax.ShapeDtypeStruct((NPAD, HALF), jnp.float32)),
    )(ua, ub, za, zb, udeg, b_prev.reshape(1, HID), W)


def _tc_fin(ua, ub, za, zb, udeg, b_prev):
    return pl.pallas_call(
        _fin_body,
        grid=(GRID,),
        in_specs=[_node_spec(), _node_spec(), _node_spec(), _node_spec(),
                  _node_spec(),
                  _full_spec((1, HID))],
        out_specs=_full_spec((1, HID)),
        out_shape=jax.ShapeDtypeStruct((1, HID), jnp.float32),
    )(ua, ub, za, zb, udeg, b_prev.reshape(1, HID))


def _head_body(sh_ref, sa_ref, hf_ref, af_ref, w_ref, b_ref, out_ref):
    inv_n = jnp.float32(1.0 / N)
    comb = jnp.concatenate(
        [sh_ref[...] * inv_n, hf_ref[...], sa_ref[...] * inv_n, af_ref[...]],
        axis=1)
    out_ref[...] = jnp.dot(comb, w_ref[...],
                           preferred_element_type=jnp.float32) + b_ref[...]


def _tc_head(sum_h, sum_a, hf, af, fc_W, fc_b):
    gf = hf.shape[0]
    return pl.pallas_call(
        _head_body,
        out_shape=jax.ShapeDtypeStruct((1, NUM_CLASSES), jnp.float32),
    )(sum_h.reshape(1, HID), sum_a.reshape(1, HID),
      hf.reshape(1, gf), af.reshape(1, gf),
      fc_W, fc_b.reshape(1, NUM_CLASSES))


def _prep_edges(edge_index, edge_weight):
    pad = PAD_E - E
    src = jnp.concatenate([edge_index[0], jnp.zeros((pad,), edge_index.dtype)])
    dst = jnp.concatenate([edge_index[1], jnp.zeros((pad,), edge_index.dtype)])
    ew = jnp.concatenate([edge_weight, jnp.zeros((pad,), edge_weight.dtype)])
    return (src.reshape(ROWS, SUB), dst.reshape(ROWS, SUB), ew.reshape(ROWS, SUB))


def _forward_graph(x, edge_index, edge_weight, conv_Ws, conv_bs):
    src2, dst2, ew2 = _prep_edges(edge_index, edge_weight)
    x = jnp.concatenate([x, jnp.zeros((NPAD - N, x.shape[1]), x.dtype)])
    ones0 = jnp.zeros((NPAD, HALF), jnp.float32).at[:, 0].set(1.0)
    udeg, _ = _edge_pass(ones0, ones0, src2, dst2, ew2)

    za, zb = _tc_layer0(x, conv_Ws[0], udeg)
    for l in range(1, len(conv_Ws)):
        ua, ub = _edge_pass(za, zb, src2, dst2, ew2)
        za, zb = _tc_mid(ua, ub, za, zb, udeg, conv_bs[l - 1], conv_Ws[l])
    ua, ub = _edge_pass(za, zb, src2, dst2, ew2)
    return _tc_fin(ua, ub, za, zb, udeg, conv_bs[-1])


def kernel(home_x, home_edge_index, home_edge_weight, away_x, away_edge_index,
           away_edge_weight, home_features, away_features, num_home_nodes,
           num_away_nodes, conv_Ws, conv_bs, fc_W, fc_b):
    sum_h = _forward_graph(home_x, home_edge_index, home_edge_weight,
                           conv_Ws, conv_bs)
    sum_a = _forward_graph(away_x, away_edge_index, away_edge_weight,
                           conv_Ws, conv_bs)
    return _tc_head(sum_h, sum_a, home_features, away_features, fc_W, fc_b)


# speedup vs baseline: 1.1315x; 1.1315x over previous
import functools

import jax
import jax.numpy as jnp
from jax import lax
from jax.experimental import pallas as pl
from jax.experimental.pallas import tpu as pltpu
from jax.experimental.pallas import tpu_sc as plsc

N = 100000
NPAD = 100096
E = 1600000
HID = 32
HALF = 16
NUM_CLASSES = 3

SUB = 128
RPB = 2
ROWS = 12800
PAD_E = ROWS * SUB
TILE_ROWS = ROWS // 16
BLOCKS = TILE_ROWS // RPB
NPT = NPAD // 16
ZCH = 128

NB = NPT
GRID = NPAD // NB


def _edge_body(za_h, zb_h, pk_h, ew_h, ua_h, ub_h,
               p0, p1, p2, p3, e0, e1, e2, e3,
               r0, r1, r2, r3, acc,
               g0, g1, g2, g3, s0, s1, s2, s3):
    c = lax.axis_index("c")
    s = lax.axis_index("s")
    B = ((p0, r0, g0, s0, e0), (p1, r1, g1, s1, e1),
         (p2, r2, g2, s2, e2), (p3, r3, g3, s3, e3))

    def zrow(i, _):
        r0[i, :] = jnp.zeros((HALF,), jnp.float32)
        return 0
    lax.fori_loop(0, RPB * SUB, zrow, 0)
    nz = NPT // ZCH
    for k in range(nz):
        pltpu.sync_copy(r0.at[pl.ds(0, ZCH)], acc.at[pl.ds(s * NPT + k * ZCH, ZCH)])
    rem = NPT - nz * ZCH
    if rem:
        pltpu.sync_copy(r0.at[pl.ds(0, rem)], acc.at[pl.ds(s * NPT + nz * ZCH, rem)])
    plsc.subcore_barrier()

    def stage(bi, blk):
        pk, eb = B[bi][0], B[bi][4]
        rowbase = s * TILE_ROWS + blk * RPB
        pltpu.sync_copy(pk_h.at[pl.ds(rowbase, RPB)], pk)
        pltpu.sync_copy(ew_h.at[pl.ds(rowbase, RPB)], eb)

    def fire_g(bi):
        pk, rows, sem = B[bi][0], B[bi][1], B[bi][2]

        @pl.when(c == 0)
        def _():
            for j in range(RPB):
                pltpu.async_copy(za_h.at[pk.at[j, 0]], rows.at[pl.ds(j * SUB, SUB)], sem)

        @pl.when(c == 1)
        def _():
            for j in range(RPB):
                pltpu.async_copy(zb_h.at[pk.at[j, 0]], rows.at[pl.ds(j * SUB, SUB)], sem)

    def drain(sem, rows):
        pltpu.make_async_copy(za_h.at[pl.ds(0, RPB * SUB)], rows, sem).wait()

    def scale(bi):
        eb, rows = B[bi][4], B[bi][1]
        for j in range(RPB):
            def s16(t, _):
                w16 = eb[j, pl.ds(t * 16, 16)]
                base = j * SUB + t * 16
                for l in range(16):
                    rows[base + l, :] = rows[base + l, :] * w16[l]
                return 0
            lax.fori_loop(0, SUB // 16, s16, 0)

    def fire_s(bi):
        pk, rows, sem = B[bi][0], B[bi][1], B[bi][3]
        for j in range(RPB):
            pltpu.async_copy(rows.at[pl.ds(j * SUB, SUB)], acc.at[pk.at[j, 1]], sem,
                             add=True)

    stage(0, 0)
    fire_g(0)
    stage(1, 1)
    fire_g(1)

    def quad(i, _):
        for u in range(4):
            blk = i * 4 + u
            bl = (u + 2) % 4

            @pl.when(blk + 2 < BLOCKS)
            def _():
                @pl.when(blk >= 2)
                def _():
                    drain(B[bl][3], B[bl][1])
                stage(bl, blk + 2)
                fire_g(bl)

            drain(B[u][2], B[u][1])
            scale(u)
            fire_s(u)
        return 0

    lax.fori_loop(0, BLOCKS // 4, quad, 0)

    drain(B[(BLOCKS - 2) % 4][3], B[(BLOCKS - 2) % 4][1])
    drain(B[(BLOCKS - 1) % 4][3], B[(BLOCKS - 1) % 4][1])
    plsc.subcore_barrier()

    @pl.when(c == 0)
    def _():
        pltpu.sync_copy(acc.at[pl.ds(s * NPT, NPT)], ua_h.at[pl.ds(s * NPT, NPT)])

    @pl.when(c == 1)
    def _():
        pltpu.sync_copy(acc.at[pl.ds(s * NPT, NPT)], ub_h.at[pl.ds(s * NPT, NPT)])


@functools.partial(jax.jit, static_argnames=())
def _edge_pass(za, zb, pk, ew2):
    f = pl.kernel(
        _edge_body,
        mesh=plsc.VectorSubcoreMesh(core_axis_name="c", subcore_axis_name="s"),
        compiler_params=pltpu.CompilerParams(use_tc_tiling_on_sc=False),
        out_type=(jax.ShapeDtypeStruct((NPAD, HALF), jnp.float32),
                  jax.ShapeDtypeStruct((NPAD, HALF), jnp.float32)),
        scratch_types=(
            [pltpu.VMEM((RPB, 2, SUB), jnp.int32)] * 4
            + [pltpu.VMEM((RPB, SUB), jnp.float32)] * 4
            + [pltpu.VMEM((RPB * SUB, HALF), jnp.float32)] * 4
            + [pltpu.VMEM_SHARED((NPAD, HALF), jnp.float32)]
            + [pltpu.SemaphoreType.DMA] * 8
        ),
    )
    return f(za, zb, pk, ew2)


def _dinv_of(udeg_blk):
    return lax.rsqrt(udeg_blk[:, 0:1] + 1.0)


def _layer0_body(x_ref, w_ref, udeg_ref, za_ref, zb_ref):
    dv = _dinv_of(udeg_ref[...])
    xw = jnp.dot(x_ref[...], w_ref[...], preferred_element_type=jnp.float32)
    z = xw * dv
    za_ref[...] = z[:, :HALF]
    zb_ref[...] = z[:, HALF:]


def _mid_body(ua_ref, ub_ref, za_ref, zb_ref, udeg_ref, bprev_ref, w_ref,
              za_o, zb_o):
    dv = _dinv_of(udeg_ref[...])
    u_plus_z = jnp.concatenate(
        [ua_ref[...] + za_ref[...], ub_ref[...] + zb_ref[...]], axis=1)
    x = jax.nn.relu(dv * u_plus_z + bprev_ref[...])
    z = dv * jnp.dot(x, w_ref[...], preferred_element_type=jnp.float32)
    za_o[...] = z[:, :HALF]
    zb_o[...] = z[:, HALF:]


def _fin_body(ua_ref, ub_ref, za_ref, zb_ref, udeg_ref, bprev_ref, sum_ref):
    dv = _dinv_of(udeg_ref[...])
    u_plus_z = jnp.concatenate(
        [ua_ref[...] + za_ref[...], ub_ref[...] + zb_ref[...]], axis=1)
    x = jax.nn.relu(dv * u_plus_z + bprev_ref[...])
    row = (lax.broadcasted_iota(jnp.int32, (NB, 1), 0)
           + pl.program_id(0) * NB)
    x = jnp.where(row < N, x, 0.0)

    @pl.when(pl.program_id(0) == 0)
    def _():
        sum_ref[...] = jnp.zeros_like(sum_ref)

    sum_ref[...] += jnp.sum(x, axis=0, keepdims=True)


def _node_spec():
    return pl.BlockSpec((NB, HALF), lambda i: (i, 0))


def _full_spec(shape):
    return pl.BlockSpec(shape, lambda i: tuple(0 for _ in shape))


def _tc_layer0(x0, W0, udeg):
    return pl.pallas_call(
        _layer0_body,
        grid=(GRID,),
        in_specs=[pl.BlockSpec((NB, 4), lambda i: (i, 0)),
                  _full_spec((4, HID)),
                  _node_spec()],
        out_specs=(_node_spec(), _node_spec()),
        out_shape=(jax.ShapeDtypeStruct((NPAD, HALF), jnp.float32),
                   jax.ShapeDtypeStruct((NPAD, HALF), jnp.float32)),
    )(x0, W0, udeg)


def _tc_mid(ua, ub, za, zb, udeg, b_prev, W):
    return pl.pallas_call(
        _mid_body,
        grid=(GRID,),
        in_specs=[_node_spec(), _node_spec(), _node_spec(), _node_spec(),
                  _node_spec(),
                  _full_spec((1, HID)),
                  _full_spec((HID, HID))],
        out_specs=(_node_spec(), _node_spec()),
        out_shape=(jax.ShapeDtypeStruct((NPAD, HALF), jnp.float32),
                   jax.ShapeDtypeStruct((NPAD, HALF), jnp.float32)),
    )(ua, ub, za, zb, udeg, b_prev.reshape(1, HID), W)


def _tc_fin(ua, ub, za, zb, udeg, b_prev):
    return pl.pallas_call(
        _fin_body,
        grid=(GRID,),
        in_specs=[_node_spec(), _node_spec(), _node_spec(), _node_spec(),
                  _node_spec(),
                  _full_spec((1, HID))],
        out_specs=_full_spec((1, HID)),
        out_shape=jax.ShapeDtypeStruct((1, HID), jnp.float32),
    )(ua, ub, za, zb, udeg, b_prev.reshape(1, HID))


def _head_body(sh_ref, sa_ref, hf_ref, af_ref, w_ref, b_ref, out_ref):
    inv_n = jnp.float32(1.0 / N)
    comb = jnp.concatenate(
        [sh_ref[...] * inv_n, hf_ref[...], sa_ref[...] * inv_n, af_ref[...]],
        axis=1)
    out_ref[...] = jnp.dot(comb, w_ref[...],
                           preferred_element_type=jnp.float32) + b_ref[...]


def _tc_head(sum_h, sum_a, hf, af, fc_W, fc_b):
    gf = hf.shape[0]
    return pl.pallas_call(
        _head_body,
        out_shape=jax.ShapeDtypeStruct((1, NUM_CLASSES), jnp.float32),
    )(sum_h.reshape(1, HID), sum_a.reshape(1, HID),
      hf.reshape(1, gf), af.reshape(1, gf),
      fc_W, fc_b.reshape(1, NUM_CLASSES))


def _prep_edges(edge_index, edge_weight):
    pad = PAD_E - E
    src = jnp.concatenate([edge_index[0], jnp.zeros((pad,), edge_index.dtype)])
    dst = jnp.concatenate([edge_index[1], jnp.zeros((pad,), edge_index.dtype)])
    ew = jnp.concatenate([edge_weight, jnp.zeros((pad,), edge_weight.dtype)])
    return (jnp.stack([src.reshape(ROWS, SUB), dst.reshape(ROWS, SUB)], axis=1),
            ew.reshape(ROWS, SUB))


def _forward_graph(x, edge_index, edge_weight, conv_Ws, conv_bs):
    pk, ew2 = _prep_edges(edge_index, edge_weight)
    x = jnp.concatenate([x, jnp.zeros((NPAD - N, x.shape[1]), x.dtype)])
    ones0 = jnp.zeros((NPAD, HALF), jnp.float32).at[:, 0].set(1.0)
    udeg, _ = _edge_pass(ones0, ones0, pk, ew2)

    za, zb = _tc_layer0(x, conv_Ws[0], udeg)
    for l in range(1, len(conv_Ws)):
        ua, ub = _edge_pass(za, zb, pk, ew2)
        za, zb = _tc_mid(ua, ub, za, zb, udeg, conv_bs[l - 1], conv_Ws[l])
    ua, ub = _edge_pass(za, zb, pk, ew2)
    return _tc_fin(ua, ub, za, zb, udeg, conv_bs[-1])


def kernel(home_x, home_edge_index, home_edge_weight, away_x, away_edge_index,
           away_edge_weight, home_features, away_features, num_home_nodes,
           num_away_nodes, conv_Ws, conv_bs, fc_W, fc_b):
    sum_h = _forward_graph(home_x, home_edge_index, home_edge_weight,
                           conv_Ws, conv_bs)
    sum_a = _forward_graph(away_x, away_edge_index, away_edge_weight,
                           conv_Ws, conv_bs)
    return _tc_head(sum_h, sum_a, home_features, away_features, fc_W, fc_b)
